# baseline (speedup 1.0000x reference)
import functools

import jax
import jax.numpy as jnp
from jax import lax
from jax.experimental import pallas as pl
from jax.experimental.pallas import tpu as pltpu
from jax.experimental.pallas import tpu_sc as plsc

VOCAB_SIZE = 1000000
D_EMBED = 64
D_HIDDEN = 256
N_BATCH = 16384

_NC = 2
_NS = 16
_NW = _NC * _NS
_L = 16
_NWIN = (VOCAB_SIZE + 127) // 128
_WIN_LO = _NWIN // _NW
_WIN_EXTRA = _NWIN - _WIN_LO * _NW
_CHUNK_LOOP = 126
_NGROUP = N_BATCH // _L
_MCAP = N_BATCH + 256 * _L
_RING = 128


def _sc_gather_scan(idx, table_t):
    mesh = plsc.VectorSubcoreMesh(core_axis_name="c", subcore_axis_name="s")

    @functools.partial(
        pl.kernel,
        mesh=mesh,
        out_type=jax.ShapeDtypeStruct((N_BATCH, D_EMBED), jnp.float32),
        compiler_params=pltpu.CompilerParams(needs_layout_passes=False),
        scratch_types=[
            pltpu.VMEM((N_BATCH,), jnp.int32),
            pltpu.VMEM((256,), jnp.int32),
            pltpu.VMEM((256,), jnp.int32),
            pltpu.VMEM((_MCAP,), jnp.int32),
            pltpu.VMEM((_MCAP,), jnp.int32),
            pltpu.VMEM((3, D_EMBED, 256), jnp.float32),
            pltpu.VMEM((_RING, D_EMBED), jnp.float32),
            pltpu.SMEM((256,), jnp.int32),
            pltpu.SMEM((256,), jnp.int32),
            pltpu.SemaphoreType.DMA,
            pltpu.SemaphoreType.DMA,
            pltpu.SemaphoreType.DMA,
            pltpu.SemaphoreType.DMA,
        ],
    )
    def gather_kernel(idx_hbm, tbl_hbm, out_hbm, idxv, cnt_v, nxt_v,
                      m_idx, m_pos, wbuf, rstage, base_s, cnt_s,
                      sem0, sem1, sem2, wsem):
        wid = lax.axis_index("s") * _NC + lax.axis_index("c")
        w0 = wid * _WIN_LO + jnp.minimum(wid, _WIN_EXTRA)
        nwin = _WIN_LO + (wid < _WIN_EXTRA).astype(jnp.int32)

        wsems = (sem0, sem1, sem2)

        def fire_chunk(c, sub):
            col = jnp.where(2 * c < nwin, (w0 + 2 * c) * 128, 0)
            col = pl.multiple_of(col, 128)
            pltpu.async_copy(tbl_hbm.at[:, pl.ds(col, 256)],
                             wbuf.at[sub], wsems[sub])

        for sub in range(3):
            fire_chunk(jnp.int32(sub), sub)

        pltpu.sync_copy(idx_hbm, idxv)

        zeros16 = jnp.zeros((_L,), jnp.int32)
        ones16 = jnp.ones((_L,), jnp.int32)
        iota16 = lax.iota(jnp.int32, _L)
        for i in range(16):
            cnt_v[pl.ds(_L * i, _L)] = zeros16

        def pass_a(g, _):
            v = idxv[pl.ds(g * _L, _L)]
            wr = (v >> 7) - w0
            m = (wr >= 0) & (wr < nwin)
            plsc.addupdate_scatter(cnt_v, [wr], ones16, mask=m)
            return 0

        lax.fori_loop(0, _NGROUP, pass_a, 0)

        run = jnp.int32(0)
        for i in range(16):
            c16 = cnt_v[pl.ds(_L * i, _L)]
            p16 = (c16 + 15) & jnp.int32(-16)
            s16 = plsc.cumsum(p16)
            excl = s16 - p16 + run
            nxt_v[pl.ds(_L * i, _L)] = excl
            for lane in range(16):
                base_s[_L * i + lane] = excl[lane]
                cnt_s[_L * i + lane] = c16[lane]
            run = excl[15] + p16[15]

        cal, _ = plsc.scan_count(zeros16)
        adj = cal[0]

        def pass_b(g, _):
            v = idxv[pl.ds(g * _L, _L)]
            wr = (v >> 7) - w0
            m = (wr >= 0) & (wr < nwin)
            b16 = plsc.load_gather(nxt_v, [wr], mask=m)
            ordn, _last = plsc.scan_count(wr, mask=m)
            slot = b16 + ordn - adj
            plsc.store_scatter(m_idx, [slot], v, mask=m)
            plsc.store_scatter(m_pos, [slot], g * _L + iota16, mask=m)
            plsc.addupdate_scatter(nxt_v, [wr], ones16, mask=m)
            return 0

        lax.fori_loop(0, _NGROUP, pass_b, 0)

        def process_window(w, sub, hoff, carry):
            fc, dr = carry
            cnt = cnt_s[w]
            b0 = base_s[w]
            ngr = (cnt + 15) >> 4

            def grp(j, c2):
                fc, dr = c2
                mi = m_idx[pl.ds(b0 + _L * j, _L)]
                pp = m_pos[pl.ds(b0 + _L * j, _L)]
                active = jnp.minimum(jnp.int32(_L), cnt - _L * j)
                need = jnp.maximum(jnp.int32(0), (fc - dr) + active - 64)

                def dwait(i, x):
                    pltpu.make_async_copy(rstage.at[pl.ds(0, 1)],
                                          out_hbm.at[pl.ds(0, 1)],
                                          wsem).wait()
                    return x

                lax.fori_loop(0, need, dwait, 0)
                for lane in range(16):
                    @pl.when(lane < active)
                    def _():
                        vcol = jnp.full((_L,), (mi[lane] & 127) + hoff,
                                        jnp.int32)
                        r = (fc + lane) & (_RING - 1)
                        for q in range(D_EMBED // _L):
                            col = plsc.load_gather(
                                wbuf.at[sub], [q * _L + iota16, vcol])
                            rstage[r, pl.ds(_L * q, _L)] = col
                        pltpu.async_copy(rstage.at[pl.ds(r, 1)],
                                         out_hbm.at[pl.ds(pp[lane], 1)],
                                         wsem)
                return (fc + active, dr + need)

            return lax.fori_loop(0, ngr, grp, (fc, dr))

        def duo(q, carry):
            for sub in range(3):
                c = q * 3 + sub
                pltpu.make_async_copy(tbl_hbm.at[:, pl.ds(0, 256)],
                                      wbuf.at[sub], wsems[sub]).wait()
                for h in range(2):
                    carry = process_window(2 * c + h, sub, 128 * h, carry)

                @pl.when(q < _CHUNK_LOOP // 3 - 1)
                def _():
                    fire_chunk(c + 3, sub)
            return carry

        fc, dr = lax.fori_loop(0, _CHUNK_LOOP // 3, duo, (jnp.int32(0),
                                                          jnp.int32(0)))

        def final_drain(i, x):
            pltpu.make_async_copy(rstage.at[pl.ds(0, 1)],
                                  out_hbm.at[pl.ds(0, 1)], wsem).wait()
            return x

        lax.fori_loop(0, fc - dr, final_drain, 0)

    return gather_kernel(idx, table_t)


def _mlp_body(x_ref, w1_ref, b1_ref, w2_ref, b2_ref, o_ref):
    h = jnp.dot(x_ref[...], w1_ref[...], preferred_element_type=jnp.float32)
    h = jnp.maximum(h + b1_ref[...], 0.0)
    o = jnp.dot(h, w2_ref[...], preferred_element_type=jnp.float32)
    o_ref[...] = jnp.maximum(o + b2_ref[...], 0.0)


def _mlp(x, W1, b1, W2, b2):
    blk = 2048
    return pl.pallas_call(
        _mlp_body,
        grid=(N_BATCH // blk,),
        in_specs=[
            pl.BlockSpec((blk, D_EMBED), lambda i: (i, 0)),
            pl.BlockSpec((D_EMBED, D_HIDDEN), lambda i: (0, 0)),
            pl.BlockSpec((1, D_HIDDEN), lambda i: (0, 0)),
            pl.BlockSpec((D_HIDDEN, D_EMBED), lambda i: (0, 0)),
            pl.BlockSpec((1, D_EMBED), lambda i: (0, 0)),
        ],
        out_specs=pl.BlockSpec((blk, D_EMBED), lambda i: (i, 0)),
        out_shape=jax.ShapeDtypeStruct((N_BATCH, D_EMBED), jnp.float32),
    )(x, W1, b1.reshape(1, D_HIDDEN), W2, b2.reshape(1, D_EMBED))


def kernel(inputs, embedding, W1, b1, W2, b2):
    x = _sc_gather_scan(inputs, embedding.T)
    return _mlp(x, W1, b1, W2, b2)


# speedup vs baseline: 1.8757x; 1.8757x over previous
import functools

import jax
import jax.numpy as jnp
from jax import lax
from jax.experimental import pallas as pl
from jax.experimental.pallas import tpu as pltpu
from jax.experimental.pallas import tpu_sc as plsc

VOCAB_SIZE = 1000000
D_EMBED = 64
D_HIDDEN = 256
N_BATCH = 16384

_NC = 2
_NS = 16
_NW = _NC * _NS
_L = 16
_NWIN = (VOCAB_SIZE + 127) // 128
_WIN_LO = _NWIN // _NW
_WIN_EXTRA = _NWIN - _WIN_LO * _NW
_WIN_LOOP = 248
_NGROUP = N_BATCH // _L
_MCAP = N_BATCH + 256 * _L
_RING = 128


def _sc_gather_scan(idx, table_t):
    mesh = plsc.VectorSubcoreMesh(core_axis_name="c", subcore_axis_name="s")

    @functools.partial(
        pl.kernel,
        mesh=mesh,
        out_type=jax.ShapeDtypeStruct((N_BATCH, D_EMBED), jnp.float32),
        compiler_params=pltpu.CompilerParams(needs_layout_passes=False),
        scratch_types=[
            pltpu.VMEM((N_BATCH,), jnp.int32),
            pltpu.VMEM((256,), jnp.int32),
            pltpu.VMEM((256,), jnp.int32),
            pltpu.VMEM((_MCAP,), jnp.int32),
            pltpu.VMEM((_MCAP,), jnp.int32),
            pltpu.VMEM((4, D_EMBED, 128), jnp.float32),
            pltpu.VMEM((_RING, D_EMBED), jnp.float32),
            pltpu.SMEM((256,), jnp.int32),
            pltpu.SMEM((256,), jnp.int32),
            pltpu.SemaphoreType.DMA,
            pltpu.SemaphoreType.DMA,
            pltpu.SemaphoreType.DMA,
            pltpu.SemaphoreType.DMA,
            pltpu.SemaphoreType.DMA,
        ],
    )
    def gather_kernel(idx_hbm, tbl_hbm, out_hbm, idxv, cnt_v, nxt_v,
                      m_idx, m_pos, wbuf, rstage, base_s, cnt_s,
                      sem0, sem1, sem2, sem3, wsem):
        wid = lax.axis_index("s") * _NC + lax.axis_index("c")
        w0 = wid * _WIN_LO + jnp.minimum(wid, _WIN_EXTRA)
        nwin = _WIN_LO + (wid < _WIN_EXTRA).astype(jnp.int32)

        wsems = (sem0, sem1, sem2, sem3)

        def fire_window(w, sub):
            col = jnp.where(w < nwin, (w0 + w) * 128, 0)
            col = pl.multiple_of(col, 128)
            pltpu.async_copy(tbl_hbm.at[:, pl.ds(col, 128)],
                             wbuf.at[sub], wsems[sub])

        for sub in range(4):
            fire_window(jnp.int32(sub), sub)

        pltpu.sync_copy(idx_hbm, idxv)

        zeros16 = jnp.zeros((_L,), jnp.int32)
        ones16 = jnp.ones((_L,), jnp.int32)
        iota16 = lax.iota(jnp.int32, _L)
        for i in range(16):
            cnt_v[pl.ds(_L * i, _L)] = zeros16

        def pass_a(g, _):
            v = idxv[pl.ds(g * _L, _L)]
            wr = (v >> 7) - w0
            m = (wr >= 0) & (wr < nwin)
            plsc.addupdate_scatter(cnt_v, [wr], ones16, mask=m)
            return 0

        lax.fori_loop(0, _NGROUP, pass_a, 0)

        run = jnp.int32(0)
        for i in range(16):
            c16 = cnt_v[pl.ds(_L * i, _L)]
            p16 = (c16 + 15) & jnp.int32(-16)
            s16 = plsc.cumsum(p16)
            excl = s16 - p16 + run
            nxt_v[pl.ds(_L * i, _L)] = excl
            for lane in range(16):
                base_s[_L * i + lane] = excl[lane]
                cnt_s[_L * i + lane] = c16[lane]
            run = excl[15] + p16[15]

        cal, _ = plsc.scan_count(zeros16)
        adj = cal[0]

        def pass_b(g, _):
            v = idxv[pl.ds(g * _L, _L)]
            wr = (v >> 7) - w0
            m = (wr >= 0) & (wr < nwin)
            b16 = plsc.load_gather(nxt_v, [wr], mask=m)
            ordn, _last = plsc.scan_count(wr, mask=m)
            slot = b16 + ordn - adj
            plsc.store_scatter(m_idx, [slot], v, mask=m)
            plsc.store_scatter(m_pos, [slot], g * _L + iota16, mask=m)
            plsc.addupdate_scatter(nxt_v, [wr], ones16, mask=m)
            return 0

        lax.fori_loop(0, _NGROUP, pass_b, 0)

        def process_window(w, sub, carry):
            fc, dr = carry
            cnt = cnt_s[w]
            b0 = base_s[w]
            ngr = (cnt + 15) >> 4

            def grp(j, c2):
                fc, dr = c2
                mi = m_idx[pl.ds(b0 + _L * j, _L)]
                pp = m_pos[pl.ds(b0 + _L * j, _L)]
                active = jnp.minimum(jnp.int32(_L), cnt - _L * j)
                need = jnp.maximum(jnp.int32(0), (fc - dr) + active - 64)

                def dwait(i, x):
                    pltpu.make_async_copy(rstage.at[pl.ds(0, 1)],
                                          out_hbm.at[pl.ds(0, 1)],
                                          wsem).wait()
                    return x

                lax.fori_loop(0, need, dwait, 0)
                for lane in range(16):
                    @pl.when(lane < active)
                    def _():
                        vcol = jnp.full((_L,), mi[lane] & 127, jnp.int32)
                        r = (fc + lane) & (_RING - 1)
                        for q in range(D_EMBED // _L):
                            col = plsc.load_gather(
                                wbuf.at[sub], [q * _L + iota16, vcol])
                            rstage[r, pl.ds(_L * q, _L)] = col
                        pltpu.async_copy(rstage.at[pl.ds(r, 1)],
                                         out_hbm.at[pl.ds(pp[lane], 1)],
                                         wsem)
                return (fc + active, dr + need)

            return lax.fori_loop(0, ngr, grp, (fc, dr))

        def quad(q, carry):
            for sub in range(4):
                w = q * 4 + sub
                pltpu.make_async_copy(tbl_hbm.at[:, pl.ds(0, 128)],
                                      wbuf.at[sub], wsems[sub]).wait()
                carry = process_window(w, sub, carry)

                @pl.when(q < _WIN_LOOP // 4 - 1)
                def _():
                    fire_window(w + 4, sub)
            return carry

        fc, dr = lax.fori_loop(0, _WIN_LOOP // 4, quad, (jnp.int32(0),
                                                         jnp.int32(0)))

        def final_drain(i, x):
            pltpu.make_async_copy(rstage.at[pl.ds(0, 1)],
                                  out_hbm.at[pl.ds(0, 1)], wsem).wait()
            return x

        lax.fori_loop(0, fc - dr, final_drain, 0)

    return gather_kernel(idx, table_t)


def _mlp_body(x_ref, w1_ref, b1_ref, w2_ref, b2_ref, o_ref):
    h = jnp.dot(x_ref[...], w1_ref[...], preferred_element_type=jnp.float32)
    h = jnp.maximum(h + b1_ref[...], 0.0)
    o = jnp.dot(h, w2_ref[...], preferred_element_type=jnp.float32)
    o_ref[...] = jnp.maximum(o + b2_ref[...], 0.0)


def _mlp(x, W1, b1, W2, b2):
    blk = 2048
    return pl.pallas_call(
        _mlp_body,
        grid=(N_BATCH // blk,),
        in_specs=[
            pl.BlockSpec((blk, D_EMBED), lambda i: (i, 0)),
            pl.BlockSpec((D_EMBED, D_HIDDEN), lambda i: (0, 0)),
            pl.BlockSpec((1, D_HIDDEN), lambda i: (0, 0)),
            pl.BlockSpec((D_HIDDEN, D_EMBED), lambda i: (0, 0)),
            pl.BlockSpec((1, D_EMBED), lambda i: (0, 0)),
        ],
        out_specs=pl.BlockSpec((blk, D_EMBED), lambda i: (i, 0)),
        out_shape=jax.ShapeDtypeStruct((N_BATCH, D_EMBED), jnp.float32),
    )(x, W1, b1.reshape(1, D_HIDDEN), W2, b2.reshape(1, D_EMBED))


def kernel(inputs, embedding, W1, b1, W2, b2):
    x = _sc_gather_scan(inputs, embedding.T)
    return _mlp(x, W1, b1, W2, b2)


# speedup vs baseline: 2.1044x; 1.1219x over previous
import functools

import jax
import jax.numpy as jnp
from jax import lax
from jax.experimental import pallas as pl
from jax.experimental.pallas import tpu as pltpu
from jax.experimental.pallas import tpu_sc as plsc

VOCAB_SIZE = 1000000
D_EMBED = 64
D_HIDDEN = 256
N_BATCH = 16384

_NC = 2
_NS = 16
_NW = _NC * _NS
_L = 16
_NWIN = (VOCAB_SIZE + 127) // 128
_WIN_LO = _NWIN // _NW
_WIN_EXTRA = _NWIN - _WIN_LO * _NW
_WIN_LOOP = 248
_NGROUP = N_BATCH // _L
_MCAP = N_BATCH + 256 * _L
_RING = 128


def _sc_gather_scan(idx, table_t):
    mesh = plsc.VectorSubcoreMesh(core_axis_name="c", subcore_axis_name="s")

    @functools.partial(
        pl.kernel,
        mesh=mesh,
        out_type=jax.ShapeDtypeStruct((N_BATCH, D_EMBED), jnp.float32),
        compiler_params=pltpu.CompilerParams(needs_layout_passes=False),
        scratch_types=[
            pltpu.VMEM((N_BATCH,), jnp.int32),
            pltpu.VMEM((256,), jnp.int32),
            pltpu.VMEM((256,), jnp.int32),
            pltpu.VMEM((_MCAP,), jnp.int32),
            pltpu.VMEM((_MCAP,), jnp.int32),
            pltpu.VMEM((4, D_EMBED, 128), jnp.float32),
            pltpu.VMEM((_RING, D_EMBED), jnp.float32),
            pltpu.SMEM((256,), jnp.int32),
            pltpu.SMEM((256,), jnp.int32),
            pltpu.SMEM((256,), jnp.int32),
            pltpu.SemaphoreType.DMA,
            pltpu.SemaphoreType.DMA,
            pltpu.SemaphoreType.DMA,
            pltpu.SemaphoreType.DMA,
            pltpu.SemaphoreType.DMA,
        ],
    )
    def gather_kernel(idx_hbm, tbl_hbm, out_hbm, idxv, cnt_v, nxt_v,
                      m_idx, m_pos, wbuf, rstage, base_s, cnt_s, wlist_s,
                      sem0, sem1, sem2, sem3, wsem):
        wid = lax.axis_index("s") * _NC + lax.axis_index("c")
        w0 = wid * _WIN_LO + jnp.minimum(wid, _WIN_EXTRA)
        nwin = _WIN_LO + (wid < _WIN_EXTRA).astype(jnp.int32)

        wsems = (sem0, sem1, sem2, sem3)

        def fire_window(w, sub):
            col = jnp.where(w < nwin, (w0 + w) * 128, 0)
            col = pl.multiple_of(col, 128)
            pltpu.async_copy(tbl_hbm.at[:, pl.ds(col, 128)],
                             wbuf.at[sub], wsems[sub])

        for sub in range(4):
            wlist_s[sub] = jnp.int32(sub)
            fire_window(jnp.int32(sub), sub)

        pltpu.sync_copy(idx_hbm, idxv)

        zeros16 = jnp.zeros((_L,), jnp.int32)
        ones16 = jnp.ones((_L,), jnp.int32)
        iota16 = lax.iota(jnp.int32, _L)
        for i in range(16):
            cnt_v[pl.ds(_L * i, _L)] = zeros16

        def pass_a(g, _):
            v = idxv[pl.ds(g * _L, _L)]
            wr = (v >> 7) - w0
            m = (wr >= 0) & (wr < nwin)
            plsc.addupdate_scatter(cnt_v, [wr], ones16, mask=m)
            return 0

        lax.fori_loop(0, _NGROUP, pass_a, 0)

        run = jnp.int32(0)
        for i in range(16):
            c16 = cnt_v[pl.ds(_L * i, _L)]
            p16 = (c16 + 15) & jnp.int32(-16)
            s16 = plsc.cumsum(p16)
            excl = s16 - p16 + run
            nxt_v[pl.ds(_L * i, _L)] = excl
            for lane in range(16):
                base_s[_L * i + lane] = excl[lane]
                cnt_s[_L * i + lane] = c16[lane]
            run = excl[15] + p16[15]

        cal, _ = plsc.scan_count(zeros16)
        adj = cal[0]

        def pass_b(g, _):
            v = idxv[pl.ds(g * _L, _L)]
            wr = (v >> 7) - w0
            m = (wr >= 0) & (wr < nwin)
            b16 = plsc.load_gather(nxt_v, [wr], mask=m)
            ordn, _last = plsc.scan_count(wr, mask=m)
            slot = b16 + ordn - adj
            plsc.store_scatter(m_idx, [slot], v, mask=m)
            plsc.store_scatter(m_pos, [slot], g * _L + iota16, mask=m)
            plsc.addupdate_scatter(nxt_v, [wr], ones16, mask=m)
            return 0

        lax.fori_loop(0, _NGROUP, pass_b, 0)

        def compact(i, nl):
            wlist_s[nl] = i
            return nl + (cnt_s[i] > 0).astype(jnp.int32)

        nl = lax.fori_loop(4, nwin, compact, jnp.int32(4))
        nl_pad = (nl + 3) & jnp.int32(-4)

        def pad(i, x):
            wlist_s[i] = nwin
            return x

        lax.fori_loop(nl, nl_pad, pad, 0)

        def process_window(w, sub, carry):
            fc, dr = carry
            cnt = cnt_s[w]
            b0 = base_s[w]
            ngr = (cnt + 15) >> 4

            def grp(j, c2):
                fc, dr = c2
                mi = m_idx[pl.ds(b0 + _L * j, _L)]
                pp = m_pos[pl.ds(b0 + _L * j, _L)]
                active = jnp.minimum(jnp.int32(_L), cnt - _L * j)
                need = jnp.maximum(jnp.int32(0), (fc - dr) + active - 64)

                def dwait(i, x):
                    pltpu.make_async_copy(rstage.at[pl.ds(0, 1)],
                                          out_hbm.at[pl.ds(0, 1)],
                                          wsem).wait()
                    return x

                lax.fori_loop(0, need, dwait, 0)
                for lane in range(16):
                    @pl.when(lane < active)
                    def _():
                        vcol = jnp.full((_L,), mi[lane] & 127, jnp.int32)
                        r = (fc + lane) & (_RING - 1)
                        for q in range(D_EMBED // _L):
                            col = plsc.load_gather(
                                wbuf.at[sub], [q * _L + iota16, vcol])
                            rstage[r, pl.ds(_L * q, _L)] = col
                        pltpu.async_copy(rstage.at[pl.ds(r, 1)],
                                         out_hbm.at[pl.ds(pp[lane], 1)],
                                         wsem)
                return (fc + active, dr + need)

            return lax.fori_loop(0, ngr, grp, (fc, dr))

        def quad(q, carry):
            for sub in range(4):
                i = q * 4 + sub
                pltpu.make_async_copy(tbl_hbm.at[:, pl.ds(0, 128)],
                                      wbuf.at[sub], wsems[sub]).wait()
                carry = process_window(wlist_s[i], sub, carry)

                @pl.when(i + 4 < nl_pad)
                def _():
                    fire_window(wlist_s[i + 4], sub)
            return carry

        fc, dr = lax.fori_loop(0, nl_pad // 4, quad, (jnp.int32(0),
                                                      jnp.int32(0)))

        def final_drain(i, x):
            pltpu.make_async_copy(rstage.at[pl.ds(0, 1)],
                                  out_hbm.at[pl.ds(0, 1)], wsem).wait()
            return x

        lax.fori_loop(0, fc - dr, final_drain, 0)

    return gather_kernel(idx, table_t)


def _mlp_body(x_ref, w1_ref, b1_ref, w2_ref, b2_ref, o_ref):
    h = jnp.dot(x_ref[...], w1_ref[...], preferred_element_type=jnp.float32)
    h = jnp.maximum(h + b1_ref[...], 0.0)
    o = jnp.dot(h, w2_ref[...], preferred_element_type=jnp.float32)
    o_ref[...] = jnp.maximum(o + b2_ref[...], 0.0)


def _mlp(x, W1, b1, W2, b2):
    blk = 2048
    return pl.pallas_call(
        _mlp_body,
        grid=(N_BATCH // blk,),
        in_specs=[
            pl.BlockSpec((blk, D_EMBED), lambda i: (i, 0)),
            pl.BlockSpec((D_EMBED, D_HIDDEN), lambda i: (0, 0)),
            pl.BlockSpec((1, D_HIDDEN), lambda i: (0, 0)),
            pl.BlockSpec((D_HIDDEN, D_EMBED), lambda i: (0, 0)),
            pl.BlockSpec((1, D_EMBED), lambda i: (0, 0)),
        ],
        out_specs=pl.BlockSpec((blk, D_EMBED), lambda i: (i, 0)),
        out_shape=jax.ShapeDtypeStruct((N_BATCH, D_EMBED), jnp.float32),
    )(x, W1, b1.reshape(1, D_HIDDEN), W2, b2.reshape(1, D_EMBED))


def kernel(inputs, embedding, W1, b1, W2, b2):
    x = _sc_gather_scan(inputs, embedding.T)
    return _mlp(x, W1, b1, W2, b2)


# speedup vs baseline: 2.1107x; 1.0030x over previous
import functools

import jax
import jax.numpy as jnp
from jax import lax
from jax.experimental import pallas as pl
from jax.experimental.pallas import tpu as pltpu
from jax.experimental.pallas import tpu_sc as plsc

VOCAB_SIZE = 1000000
D_EMBED = 64
D_HIDDEN = 256
N_BATCH = 16384

_NC = 2
_NS = 16
_NW = _NC * _NS
_L = 16
_NWIN = (VOCAB_SIZE + 127) // 128
_WIN_LO = _NWIN // _NW
_WIN_EXTRA = _NWIN - _WIN_LO * _NW
_NGROUP = N_BATCH // _L
_MCAP = N_BATCH + 256 * _L
_RING = 128


def _sc_gather_scan(idx, table_t):
    mesh = plsc.VectorSubcoreMesh(core_axis_name="c", subcore_axis_name="s")

    @functools.partial(
        pl.kernel,
        mesh=mesh,
        out_type=jax.ShapeDtypeStruct((N_BATCH, D_EMBED), jnp.float32),
        compiler_params=pltpu.CompilerParams(needs_layout_passes=False),
        scratch_types=[
            pltpu.VMEM((N_BATCH,), jnp.int32),
            pltpu.VMEM((256,), jnp.int32),
            pltpu.VMEM((256,), jnp.int32),
            pltpu.VMEM((_MCAP,), jnp.int32),
            pltpu.VMEM((_MCAP,), jnp.int32),
            pltpu.VMEM((4, D_EMBED, 128), jnp.float32),
            pltpu.VMEM((_RING, D_EMBED), jnp.float32),
            pltpu.SMEM((256,), jnp.int32),
            pltpu.SMEM((256,), jnp.int32),
            pltpu.SMEM((256,), jnp.int32),
            pltpu.SemaphoreType.DMA,
            pltpu.SemaphoreType.DMA,
            pltpu.SemaphoreType.DMA,
            pltpu.SemaphoreType.DMA,
            pltpu.SemaphoreType.DMA,
        ],
    )
    def gather_kernel(idx_hbm, tbl_hbm, out_hbm, idxv, cnt_v, nxt_v,
                      m_idx, m_pos, wbuf, rstage, base_s, cnt_s, wlist_s,
                      sem0, sem1, sem2, sem3, wsem):
        wid = lax.axis_index("s") * _NC + lax.axis_index("c")
        w0 = wid * _WIN_LO + jnp.minimum(wid, _WIN_EXTRA)
        nwin = _WIN_LO + (wid < _WIN_EXTRA).astype(jnp.int32)

        wsems = (sem0, sem1, sem2, sem3)

        def fire_window(w, sub):
            col = jnp.where(w < nwin, (w0 + w) * 128, 0)
            col = pl.multiple_of(col, 128)
            pltpu.async_copy(tbl_hbm.at[:, pl.ds(col, 128)],
                             wbuf.at[sub], wsems[sub])

        for sub in range(4):
            wlist_s[sub] = jnp.int32(sub)
            fire_window(jnp.int32(sub), sub)

        pltpu.sync_copy(idx_hbm, idxv)

        zeros16 = jnp.zeros((_L,), jnp.int32)
        ones16 = jnp.ones((_L,), jnp.int32)
        iota16 = lax.iota(jnp.int32, _L)
        for i in range(16):
            cnt_v[pl.ds(_L * i, _L)] = zeros16

        def pass_a(g, _):
            v = idxv[pl.ds(g * _L, _L)]
            wr = (v >> 7) - w0
            m = (wr >= 0) & (wr < nwin)
            plsc.addupdate_scatter(cnt_v, [wr], ones16, mask=m)
            return 0

        lax.fori_loop(0, _NGROUP, pass_a, 0)

        run = jnp.int32(0)
        for i in range(16):
            c16 = cnt_v[pl.ds(_L * i, _L)]
            p16 = (c16 + 15) & jnp.int32(-16)
            s16 = plsc.cumsum(p16)
            excl = s16 - p16 + run
            nxt_v[pl.ds(_L * i, _L)] = excl
            for lane in range(16):
                base_s[_L * i + lane] = excl[lane]
                cnt_s[_L * i + lane] = c16[lane]
            run = excl[15] + p16[15]

        cal, _ = plsc.scan_count(zeros16)
        adj = cal[0]

        def pass_b(g, _):
            v = idxv[pl.ds(g * _L, _L)]
            wr = (v >> 7) - w0
            m = (wr >= 0) & (wr < nwin)
            b16 = plsc.load_gather(nxt_v, [wr], mask=m)
            ordn, _last = plsc.scan_count(wr, mask=m)
            slot = b16 + ordn - adj
            plsc.store_scatter(m_idx, [slot], v, mask=m)
            plsc.store_scatter(m_pos, [slot], g * _L + iota16, mask=m)
            plsc.addupdate_scatter(nxt_v, [wr], ones16, mask=m)
            return 0

        lax.fori_loop(0, _NGROUP, pass_b, 0)

        def compact(i, nl):
            wlist_s[nl] = i
            return nl + (cnt_s[i] > 0).astype(jnp.int32)

        nl = lax.fori_loop(4, nwin, compact, jnp.int32(4))
        nl_pad = (nl + 3) & jnp.int32(-4)

        def pad(i, x):
            wlist_s[i] = nwin
            return x

        lax.fori_loop(nl, nl_pad, pad, 0)

        def process_window(w, sub, carry):
            fc, dr = carry
            cnt = cnt_s[w]
            b0 = base_s[w]
            ngr = (cnt + 15) >> 4

            def grp(j, c2):
                fc, dr = c2
                mi = m_idx[pl.ds(b0 + _L * j, _L)]
                pp = m_pos[pl.ds(b0 + _L * j, _L)]
                active = jnp.minimum(jnp.int32(_L), cnt - _L * j)
                need = jnp.maximum(jnp.int32(0), (fc - dr) + active - 64)

                def dwait(i, x):
                    pltpu.make_async_copy(rstage.at[pl.ds(0, 1)],
                                          out_hbm.at[pl.ds(0, 1)],
                                          wsem).wait()
                    return x

                lax.fori_loop(0, need, dwait, 0)
                for lane in range(16):
                    @pl.when(lane < active)
                    def _():
                        vcol = jnp.full((_L,), mi[lane] & 127, jnp.int32)
                        r = (fc + lane) & (_RING - 1)
                        for q in range(D_EMBED // _L):
                            col = plsc.load_gather(
                                wbuf.at[sub], [q * _L + iota16, vcol])
                            rstage[r, pl.ds(_L * q, _L)] = col
                        pltpu.async_copy(rstage.at[pl.ds(r, 1)],
                                         out_hbm.at[pl.ds(pp[lane], 1)],
                                         wsem)
                return (fc + active, dr + need)

            return lax.fori_loop(0, ngr, grp, (fc, dr))

        def quad(q, carry):
            for sub in range(4):
                i = q * 4 + sub
                pltpu.make_async_copy(tbl_hbm.at[:, pl.ds(0, 128)],
                                      wbuf.at[sub], wsems[sub]).wait()
                carry = process_window(wlist_s[i], sub, carry)

                @pl.when(i + 4 < nl_pad)
                def _():
                    fire_window(wlist_s[i + 4], sub)
            return carry

        fc, dr = lax.fori_loop(0, nl_pad // 4, quad, (jnp.int32(0),
                                                      jnp.int32(0)))

        def final_drain(i, x):
            pltpu.make_async_copy(rstage.at[pl.ds(0, 1)],
                                  out_hbm.at[pl.ds(0, 1)], wsem).wait()
            return x

        lax.fori_loop(0, fc - dr, final_drain, 0)

    return gather_kernel(idx, table_t)


def _mlp_body(x_ref, w1_ref, b1_ref, w2_ref, b2_ref, o_ref):
    h = jnp.dot(x_ref[...], w1_ref[...], preferred_element_type=jnp.float32)
    h = jnp.maximum(h + b1_ref[...], 0.0)
    o = jnp.dot(h, w2_ref[...], preferred_element_type=jnp.float32)
    o_ref[...] = jnp.maximum(o + b2_ref[...], 0.0)


def _mlp(x, W1, b1, W2, b2):
    blk = 2048
    return pl.pallas_call(
        _mlp_body,
        grid=(N_BATCH // blk,),
        in_specs=[
            pl.BlockSpec((blk, D_EMBED), lambda i: (i, 0)),
            pl.BlockSpec((D_EMBED, D_HIDDEN), lambda i: (0, 0)),
            pl.BlockSpec((1, D_HIDDEN), lambda i: (0, 0)),
            pl.BlockSpec((D_HIDDEN, D_EMBED), lambda i: (0, 0)),
            pl.BlockSpec((1, D_EMBED), lambda i: (0, 0)),
        ],
        out_specs=pl.BlockSpec((blk, D_EMBED), lambda i: (i, 0)),
        out_shape=jax.ShapeDtypeStruct((N_BATCH, D_EMBED), jnp.float32),
    )(x, W1, b1.reshape(1, D_HIDDEN), W2, b2.reshape(1, D_EMBED))


def kernel(inputs, embedding, W1, b1, W2, b2):
    x = _sc_gather_scan(inputs, embedding.T)
    return _mlp(x, W1, b1, W2, b2)
